# baseline (speedup 1.0000x reference)
import jax
import jax.numpy as jnp
from jax.experimental import pallas as pl
from jax.experimental.pallas import tpu as pltpu

NUM_Q_HEADS = 16
V_HEAD_DIM = 128
GATE_NUM = 3
RESCALE = (V_HEAD_DIM * GATE_NUM) ** (-0.5)

_BLOCK_T = 512


def _fuse_body(com_ref, slc_ref, sw_ref, w_ref, out_ref):
    com = com_ref[...]
    slc = slc_ref[...]
    sw = sw_ref[...]
    w = w_ref[...]

    def score(g):
        p = com * w[g, 0][None]
        p += slc * w[g, 1][None]
        p += sw * w[g, 2][None]
        return p.sum(axis=-1)

    e0 = jnp.exp(score(0))
    e1 = jnp.exp(score(1))
    e2 = jnp.exp(score(2))
    inv = 1.0 / (e0 + e1 + e2)
    fused = (
        e0[..., None] * com + e1[..., None] * slc + e2[..., None] * sw
    ) * inv[..., None]
    out_ref[...] = fused.reshape(fused.shape[0] * NUM_Q_HEADS, V_HEAD_DIM)


@jax.jit
def kernel(o_com_att, o_slc_att, o_sw_att, gate_weight):
    T = o_com_att.shape[0]
    w = gate_weight.T.reshape(NUM_Q_HEADS, GATE_NUM, GATE_NUM, V_HEAD_DIM)
    w = jnp.transpose(w, (1, 2, 0, 3)) * RESCALE

    grid = (T // _BLOCK_T,)
    in_spec = pl.BlockSpec(
        (_BLOCK_T, NUM_Q_HEADS, V_HEAD_DIM), lambda i: (i, 0, 0)
    )
    w_spec = pl.BlockSpec(
        (GATE_NUM, GATE_NUM, NUM_Q_HEADS, V_HEAD_DIM), lambda i: (0, 0, 0, 0)
    )
    out_spec = pl.BlockSpec(
        (_BLOCK_T * NUM_Q_HEADS, V_HEAD_DIM), lambda i: (i, 0)
    )
    out = pl.pallas_call(
        _fuse_body,
        grid=grid,
        in_specs=[in_spec, in_spec, in_spec, w_spec],
        out_specs=out_spec,
        out_shape=jax.ShapeDtypeStruct(
            (T * NUM_Q_HEADS, V_HEAD_DIM), jnp.float32
        ),
    )(o_com_att, o_slc_att, o_sw_att, w)
    return out.reshape(T, NUM_Q_HEADS * V_HEAD_DIM)


# speedup vs baseline: 1.6757x; 1.6757x over previous
import jax
import jax.numpy as jnp
from jax.experimental import pallas as pl
from jax.experimental.pallas import tpu as pltpu

NUM_Q_HEADS = 16
V_HEAD_DIM = 128
GATE_NUM = 3
RESCALE = (V_HEAD_DIM * GATE_NUM) ** (-0.5)

_BLOCK_T = 512


def _fuse_body(com_ref, slc_ref, sw_ref, w_ref, out_ref):
    com = com_ref[...]
    slc = slc_ref[...]
    sw = sw_ref[...]
    w = w_ref[...]

    def score(g):
        p = com * w[g, 0][None]
        p += slc * w[g, 1][None]
        p += sw * w[g, 2][None]
        return p.sum(axis=-1)

    e0 = jnp.exp(score(0))
    e1 = jnp.exp(score(1))
    e2 = jnp.exp(score(2))
    inv = 1.0 / (e0 + e1 + e2)
    fused = (
        e0[..., None] * com + e1[..., None] * slc + e2[..., None] * sw
    ) * inv[..., None]
    out_ref[...] = fused.reshape(fused.shape[0], NUM_Q_HEADS * V_HEAD_DIM)


@jax.jit
def kernel(o_com_att, o_slc_att, o_sw_att, gate_weight):
    T = o_com_att.shape[0]
    w = gate_weight.T.reshape(NUM_Q_HEADS, GATE_NUM, GATE_NUM, V_HEAD_DIM)
    w = jnp.transpose(w, (1, 2, 0, 3)) * RESCALE

    grid = (T // _BLOCK_T,)
    in_spec = pl.BlockSpec(
        (_BLOCK_T, NUM_Q_HEADS, V_HEAD_DIM), lambda i: (i, 0, 0)
    )
    w_spec = pl.BlockSpec(
        (GATE_NUM, GATE_NUM, NUM_Q_HEADS, V_HEAD_DIM), lambda i: (0, 0, 0, 0)
    )
    out_spec = pl.BlockSpec(
        (_BLOCK_T, NUM_Q_HEADS * V_HEAD_DIM), lambda i: (i, 0)
    )
    return pl.pallas_call(
        _fuse_body,
        grid=grid,
        in_specs=[in_spec, in_spec, in_spec, w_spec],
        out_specs=out_spec,
        out_shape=jax.ShapeDtypeStruct(
            (T, NUM_Q_HEADS * V_HEAD_DIM), jnp.float32
        ),
    )(o_com_att, o_slc_att, o_sw_att, w)
